# baseline (speedup 1.0000x reference)
import functools
import math

import jax
import jax.numpy as jnp
from jax import lax
from jax.experimental import pallas as pl
from jax.experimental.pallas import tpu as pltpu
from jax.experimental.pallas import tpu_sc as plsc

D_DIM = 128
EMB_SCALE = math.sqrt(float(D_DIM))


def kernel(input_ids, token_type_ids, table):
    B, L = input_ids.shape
    N = B * L
    ids_a = input_ids.reshape(N)
    ids_b = token_type_ids.reshape(N)

    info = plsc.get_sparse_core_info()
    NC, NS = info.num_cores, info.num_subcores
    NW = NC * NS
    assert N % NW == 0
    per_w = N // NW
    C = 160
    C2 = 2 * C
    assert per_w % (2 * C) == 0
    n_chunks = per_w // C
    H = n_chunks // 2

    ids2 = jnp.stack(
        [ids_a.reshape(NW, n_chunks, C), ids_b.reshape(NW, n_chunks, C)],
        axis=2,
    ).reshape(NW * n_chunks * C2)

    mesh = plsc.VectorSubcoreMesh(core_axis_name="c", subcore_axis_name="s")

    @functools.partial(
        pl.kernel,
        mesh=mesh,
        out_type=jax.ShapeDtypeStruct((N, D_DIM), jnp.float32),
        scratch_types=[
            pltpu.VMEM((C,), jnp.int32),
            pltpu.VMEM((C,), jnp.int32),
            pltpu.VMEM((C,), jnp.int32),
            pltpu.VMEM((C,), jnp.int32),
            pltpu.VMEM((C, D_DIM), jnp.float32),
            pltpu.VMEM((C, D_DIM), jnp.float32),
            pltpu.VMEM((C, D_DIM), jnp.float32),
            pltpu.VMEM((C, D_DIM), jnp.float32),
            pltpu.VMEM((C, D_DIM), jnp.float32),
            pltpu.VMEM((C, D_DIM), jnp.float32),
            pltpu.SemaphoreType.DMA,
            pltpu.SemaphoreType.DMA,
            pltpu.SemaphoreType.DMA,
            pltpu.SemaphoreType.DMA,
        ],
    )
    def sc_embed(tab, ids_hbm, out_hbm,
                 ixa0, ixb0, ixa1, ixb1,
                 ba0, bb0, bo0, ba1, bb1, bo1,
                 sg0, sg1, so0, so1):
        wid = lax.axis_index("s") * NC + lax.axis_index("c")
        base = wid * per_w
        ibase = wid * n_chunks * C2
        IXA = (ixa0, ixa1)
        IXB = (ixb0, ixb1)
        BA = (ba0, ba1)
        BB = (bb0, bb1)
        BO = (bo0, bo1)
        SG = (sg0, sg1)
        SO = (so0, so1)

        def fetch(g, p):
            off = ibase + g * C2
            pltpu.sync_copy(ids_hbm.at[pl.ds(off, C)], IXA[p])
            pltpu.sync_copy(ids_hbm.at[pl.ds(off + C, C)], IXB[p])
            pltpu.async_copy(tab.at[IXA[p]], BA[p], SG[p])
            pltpu.async_copy(tab.at[IXB[p]], BB[p], SG[p])

        def wait_gathers(p):
            pltpu.make_async_copy(tab.at[IXA[p]], BA[p], SG[p]).wait()
            pltpu.make_async_copy(tab.at[IXB[p]], BB[p], SG[p]).wait()

        def compute(p):
            ba, bb, bo = BA[p], BB[p], BO[p]

            @plsc.parallel_loop(0, C, 1, unroll=2)
            def _(r):
                for j in range(D_DIM // 16):
                    s = pl.ds(j * 16, 16)
                    bo[r, s] = ba[r, s] * EMB_SCALE + bb[r, s]

        def put(g, p):
            pltpu.async_copy(BO[p], out_hbm.at[pl.ds(base + g * C, C)], SO[p])

        def wait_put(p):
            pltpu.make_async_copy(BO[p], out_hbm.at[pl.ds(base, C)],
                                  SO[p]).wait()

        fetch(0, 0)

        def body(h, carry):
            g0 = 2 * h
            fetch(g0 + 1, 1)
            wait_gathers(0)

            @pl.when(h > 0)
            def _():
                wait_put(0)

            compute(0)
            put(g0, 0)

            @pl.when(h < H - 1)
            def _():
                fetch(g0 + 2, 0)

            wait_gathers(1)

            @pl.when(h > 0)
            def _():
                wait_put(1)

            compute(1)
            put(g0 + 1, 1)
            return carry

        lax.fori_loop(0, H, body, 0)
        wait_put(0)
        wait_put(1)

    out = sc_embed(table, ids2)
    return out.reshape(B, L, D_DIM)


# speedup vs baseline: 1.1038x; 1.1038x over previous
import functools
import math

import jax
import jax.numpy as jnp
from jax import lax
from jax.experimental import pallas as pl
from jax.experimental.pallas import tpu as pltpu
from jax.experimental.pallas import tpu_sc as plsc

D_DIM = 128
EMB_SCALE = math.sqrt(float(D_DIM))


def kernel(input_ids, token_type_ids, table):
    B, L = input_ids.shape
    N = B * L
    ids_a = input_ids.reshape(N)
    ids_b = token_type_ids.reshape(N)

    info = plsc.get_sparse_core_info()
    NC, NS = info.num_cores, info.num_subcores
    NW = NC * NS
    assert N % NW == 0
    per_w = N // NW
    C = 160
    assert per_w % (2 * C) == 0
    n_chunks = per_w // C
    H = n_chunks // 2

    mesh = plsc.VectorSubcoreMesh(core_axis_name="c", subcore_axis_name="s")

    @functools.partial(
        pl.kernel,
        mesh=mesh,
        out_type=jax.ShapeDtypeStruct((N, D_DIM), jnp.float32),
        scratch_types=[
            pltpu.VMEM((C,), jnp.int32),
            pltpu.VMEM((C,), jnp.int32),
            pltpu.VMEM((C,), jnp.int32),
            pltpu.VMEM((C,), jnp.int32),
            pltpu.VMEM((C, D_DIM), jnp.float32),
            pltpu.VMEM((C, D_DIM), jnp.float32),
            pltpu.VMEM((C, D_DIM), jnp.float32),
            pltpu.VMEM((C, D_DIM), jnp.float32),
            pltpu.VMEM((C, D_DIM), jnp.float32),
            pltpu.VMEM((C, D_DIM), jnp.float32),
            pltpu.SemaphoreType.DMA,
            pltpu.SemaphoreType.DMA,
            pltpu.SemaphoreType.DMA,
            pltpu.SemaphoreType.DMA,
        ],
    )
    def sc_embed(tab, a_hbm, b_hbm, out_hbm,
                 ixa0, ixb0, ixa1, ixb1,
                 ba0, bb0, bo0, ba1, bb1, bo1,
                 sg0, sg1, so0, so1):
        wid = lax.axis_index("s") * NC + lax.axis_index("c")
        base = wid * per_w
        IXA = (ixa0, ixa1)
        IXB = (ixb0, ixb1)
        BA = (ba0, ba1)
        BB = (bb0, bb1)
        BO = (bo0, bo1)
        SG = (sg0, sg1)
        SO = (so0, so1)

        def fetch(g, p):
            off = base + g * C
            pltpu.sync_copy(a_hbm.at[pl.ds(off, C)], IXA[p])
            pltpu.sync_copy(b_hbm.at[pl.ds(off, C)], IXB[p])
            pltpu.async_copy(tab.at[IXA[p]], BA[p], SG[p])
            pltpu.async_copy(tab.at[IXB[p]], BB[p], SG[p])

        def wait_gathers(p):
            pltpu.make_async_copy(tab.at[IXA[p]], BA[p], SG[p]).wait()
            pltpu.make_async_copy(tab.at[IXB[p]], BB[p], SG[p]).wait()

        def compute(p):
            ba, bb, bo = BA[p], BB[p], BO[p]

            @plsc.parallel_loop(0, C, 1, unroll=2)
            def _(r):
                for j in range(D_DIM // 16):
                    s = pl.ds(j * 16, 16)
                    bo[r, s] = ba[r, s] * EMB_SCALE + bb[r, s]

        def put(g, p):
            pltpu.async_copy(BO[p], out_hbm.at[pl.ds(base + g * C, C)], SO[p])

        def wait_put(p):
            pltpu.make_async_copy(BO[p], out_hbm.at[pl.ds(base, C)],
                                  SO[p]).wait()

        fetch(0, 0)

        def body(h, carry):
            g0 = 2 * h
            fetch(g0 + 1, 1)
            wait_gathers(0)

            @pl.when(h > 0)
            def _():
                wait_put(0)

            compute(0)
            put(g0, 0)

            @pl.when(h < H - 1)
            def _():
                fetch(g0 + 2, 0)

            wait_gathers(1)

            @pl.when(h > 0)
            def _():
                wait_put(1)

            compute(1)
            put(g0 + 1, 1)
            return carry

        lax.fori_loop(0, H, body, 0)
        wait_put(0)
        wait_put(1)

    out = sc_embed(table, ids_a, ids_b)
    return out.reshape(B, L, D_DIM)
